# Initial kernel scaffold; baseline (speedup 1.0000x reference)
import functools

import jax
import jax.numpy as jnp
from jax import lax
from jax.experimental import pallas as pl
from jax.experimental.pallas import tpu as pltpu
from jax.experimental.pallas import tpu_sc as plsc

N = 10000
D = 128
E = 320000

NC = 2
NS = 16
NW = NC * NS

NP = 10240
NPT = NP // NS
K = 128
C = 80
EPT = C * K
EPAD = NW * EPT

BR = 1024
G = NP // BR

_MESH = plsc.VectorSubcoreMesh(
    core_axis_name="c", subcore_axis_name="s", num_cores=NC, num_subcores=NS)


@functools.partial(
    pl.kernel,
    out_type=jax.ShapeDtypeStruct((NC, NP, 16), jnp.float32),
    mesh=_MESH,
    scratch_types=[
        pltpu.VMEM((C, K), jnp.int32),
        pltpu.VMEM((K, 16), jnp.float32),
        pltpu.VMEM((NPT, 16), jnp.float32),
        pltpu.VMEM_SHARED((NP, 16), jnp.float32),
    ],
)
def _sc_degree(dst_hbm, out_hbm, dstbuf, ones, zbuf, acc):
    cid = lax.axis_index("c")
    sid = lax.axis_index("s")
    wid = sid * NC + cid

    def fill_ones(i, _):
        ones[i, :] = jnp.ones((16,), jnp.float32)
        return 0
    lax.fori_loop(0, K, fill_ones, 0)

    def fill_zero(i, _):
        zbuf[i, :] = jnp.zeros((16,), jnp.float32)
        return 0
    lax.fori_loop(0, NPT, fill_zero, 0)

    pltpu.sync_copy(dst_hbm.at[wid], dstbuf)
    pltpu.sync_copy(zbuf, acc.at[pl.ds(sid * NPT, NPT)])
    plsc.subcore_barrier()

    def body(j, _):
        pltpu.sync_copy(ones, acc.at[dstbuf.at[j]], add=True)
        return 0
    lax.fori_loop(0, C, body, 0)

    plsc.subcore_barrier()
    pltpu.sync_copy(acc.at[pl.ds(sid * NPT, NPT)], zbuf)
    pltpu.sync_copy(zbuf, out_hbm.at[cid, pl.ds(sid * NPT, NPT)])


@functools.partial(
    pl.kernel,
    out_type=jax.ShapeDtypeStruct((NC, NP, D), jnp.float32),
    mesh=_MESH,
    scratch_types=[
        pltpu.VMEM((C, K), jnp.int32),
        pltpu.VMEM((C, K), jnp.int32),
        pltpu.VMEM((K, D), jnp.float32),
        pltpu.VMEM_SHARED((NP, D), jnp.float32),
        pltpu.SemaphoreType.DMA,
    ],
)
def _sc_aggregate(hs_hbm, src_hbm, dst_hbm, out_hbm, srcbuf, dstbuf, rows, acc, sem):
    cid = lax.axis_index("c")
    sid = lax.axis_index("s")
    wid = sid * NC + cid

    def fill_zero(e, _):
        for c8 in range(D // 16):
            rows[e, pl.ds(c8 * 16, 16)] = jnp.zeros((16,), jnp.float32)
        return 0
    lax.fori_loop(0, K, fill_zero, 0)

    for i in range(NPT // K):
        pltpu.sync_copy(rows, acc.at[pl.ds(sid * NPT + i * K, K)])

    pltpu.sync_copy(src_hbm.at[wid], srcbuf)
    pltpu.sync_copy(dst_hbm.at[wid], dstbuf)
    plsc.subcore_barrier()

    def body(j, _):
        pltpu.async_copy(hs_hbm.at[srcbuf.at[j]], rows, sem).wait()
        pltpu.sync_copy(rows, acc.at[dstbuf.at[j]], add=True)
        return 0
    lax.fori_loop(0, C, body, 0)

    plsc.subcore_barrier()
    for i in range(NPT // K):
        pltpu.sync_copy(acc.at[pl.ds(sid * NPT + i * K, K)], rows)
        pltpu.sync_copy(rows, out_hbm.at[cid, pl.ds(sid * NPT + i * K, K)])


def _dinv(deg_ref):
    deg = deg_ref[0, :, 0:1] + deg_ref[1, :, 0:1] + 1.0
    return lax.rsqrt(deg)


def _tc1_body(x_ref, w_ref, deg_ref, hs_ref):
    dinv = _dinv(deg_ref)
    hs_ref[...] = dinv * jnp.dot(x_ref[...], w_ref[...],
                                 preferred_element_type=jnp.float32)


def _tc2_body(agg_ref, hs1_ref, deg_ref, b1_ref, w2_ref, hs2_ref):
    dinv = _dinv(deg_ref)
    tot = agg_ref[0] + agg_ref[1] + hs1_ref[...]
    h = jnp.maximum(dinv * tot + b1_ref[...], 0.0)
    hs2_ref[...] = dinv * jnp.dot(h, w2_ref[...],
                                  preferred_element_type=jnp.float32)


def _tc3_body(agg_ref, hs2_ref, deg_ref, b2_ref, out_ref):
    dinv = _dinv(deg_ref)
    tot = agg_ref[0] + agg_ref[1] + hs2_ref[...]
    out_ref[...] = dinv * tot + b2_ref[...]


_ROWS = pl.BlockSpec((BR, D), lambda i: (i, 0))
_FULL = pl.BlockSpec((D, D), lambda i: (0, 0))
_DEG = pl.BlockSpec((NC, BR, 16), lambda i: (0, i, 0))
_AGG = pl.BlockSpec((NC, BR, D), lambda i: (0, i, 0))
_BIAS = pl.BlockSpec((1, D), lambda i: (0, 0))

_tc1 = pl.pallas_call(
    _tc1_body, grid=(G,),
    in_specs=[_ROWS, _FULL, _DEG],
    out_specs=_ROWS,
    out_shape=jax.ShapeDtypeStruct((NP, D), jnp.float32),
)

_tc2 = pl.pallas_call(
    _tc2_body, grid=(G,),
    in_specs=[_AGG, _ROWS, _DEG, _BIAS, _FULL],
    out_specs=_ROWS,
    out_shape=jax.ShapeDtypeStruct((NP, D), jnp.float32),
)

_tc3 = pl.pallas_call(
    _tc3_body, grid=(G,),
    in_specs=[_AGG, _ROWS, _DEG, _BIAS],
    out_specs=_ROWS,
    out_shape=jax.ShapeDtypeStruct((NP, D), jnp.float32),
)


def kernel(x, edge_index, W1, b1, W2, b2):
    pad = EPAD - E
    src = jnp.concatenate(
        [edge_index[0], jnp.zeros((pad,), jnp.int32)]).reshape(NW, C, K)
    dst = jnp.concatenate(
        [edge_index[1], jnp.full((pad,), N, jnp.int32)]).reshape(NW, C, K)
    x_pad = jnp.pad(x, ((0, NP - N), (0, 0)))

    deg = _sc_degree(dst)
    hs1 = _tc1(x_pad, W1, deg)
    agg1 = _sc_aggregate(hs1, src, dst)
    hs2 = _tc2(agg1, hs1, deg, b1.reshape(1, D), W2)
    agg2 = _sc_aggregate(hs2, src, dst)
    out = _tc3(agg2, hs2, deg, b2.reshape(1, D))
    return out[:N]


# speedup vs baseline: 1.5103x; 1.5103x over previous
import functools

import jax
import jax.numpy as jnp
from jax import lax
from jax.experimental import pallas as pl
from jax.experimental.pallas import tpu as pltpu

N = 10000
D = 128
E = 320000

NP = 10240
BE = 1024
NBLK = 320
EPAD = NBLK * BE

BR = 1024
G = NP // BR


def _deg_body(dst_ref, deg_ref):
    @pl.when(pl.program_id(0) == 0)
    def _():
        deg_ref[...] = jnp.zeros((NP, 1), jnp.float32)

    def body(e, _):
        d = dst_ref[0, 0, e]
        deg_ref[pl.ds(d, 1), :] = deg_ref[pl.ds(d, 1), :] + 1.0
        return 0
    lax.fori_loop(0, BE, body, 0)


_k_deg = pl.pallas_call(
    _deg_body,
    grid=(NBLK,),
    in_specs=[pl.BlockSpec((1, 1, BE), lambda i: (i, 0, 0),
                           memory_space=pltpu.SMEM)],
    out_specs=pl.BlockSpec((NP, 1), lambda i: (0, 0)),
    out_shape=jax.ShapeDtypeStruct((NP, 1), jnp.float32),
)


def _agg_body(hs_ref, src_ref, dst_ref, out_ref):
    @pl.when(pl.program_id(0) == 0)
    def _():
        out_ref[...] = jnp.zeros((NP, D), jnp.float32)

    def body(e, _):
        s = src_ref[0, 0, e]
        d = dst_ref[0, 0, e]
        out_ref[pl.ds(d, 1), :] = (
            out_ref[pl.ds(d, 1), :] + hs_ref[pl.ds(s, 1), :])
        return 0
    lax.fori_loop(0, BE, body, 0)


_k_agg = pl.pallas_call(
    _agg_body,
    grid=(NBLK,),
    in_specs=[pl.BlockSpec((NP, D), lambda i: (0, 0)),
              pl.BlockSpec((1, 1, BE), lambda i: (i, 0, 0),
                           memory_space=pltpu.SMEM),
              pl.BlockSpec((1, 1, BE), lambda i: (i, 0, 0),
                           memory_space=pltpu.SMEM)],
    out_specs=pl.BlockSpec((NP, D), lambda i: (0, 0)),
    out_shape=jax.ShapeDtypeStruct((NP, D), jnp.float32),
)


def _dinv(deg_ref):
    deg = deg_ref[...] + 1.0
    return lax.rsqrt(deg)


def _tc1_body(x_ref, w_ref, deg_ref, hs_ref):
    dinv = _dinv(deg_ref)
    hs_ref[...] = dinv * jnp.dot(x_ref[...], w_ref[...],
                                 preferred_element_type=jnp.float32)


def _tc2_body(agg_ref, hs1_ref, deg_ref, b1_ref, w2_ref, hs2_ref):
    dinv = _dinv(deg_ref)
    tot = agg_ref[...] + hs1_ref[...]
    h = jnp.maximum(dinv * tot + b1_ref[...], 0.0)
    hs2_ref[...] = dinv * jnp.dot(h, w2_ref[...],
                                  preferred_element_type=jnp.float32)


def _tc3_body(agg_ref, hs2_ref, deg_ref, b2_ref, out_ref):
    dinv = _dinv(deg_ref)
    tot = agg_ref[...] + hs2_ref[...]
    out_ref[...] = dinv * tot + b2_ref[...]


_ROWS = pl.BlockSpec((BR, D), lambda i: (i, 0))
_FULL = pl.BlockSpec((D, D), lambda i: (0, 0))
_DEGS = pl.BlockSpec((BR, 1), lambda i: (i, 0))
_BIAS = pl.BlockSpec((1, D), lambda i: (0, 0))

_tc1 = pl.pallas_call(
    _tc1_body, grid=(G,),
    in_specs=[_ROWS, _FULL, _DEGS],
    out_specs=_ROWS,
    out_shape=jax.ShapeDtypeStruct((NP, D), jnp.float32),
)

_tc2 = pl.pallas_call(
    _tc2_body, grid=(G,),
    in_specs=[_ROWS, _ROWS, _DEGS, _BIAS, _FULL],
    out_specs=_ROWS,
    out_shape=jax.ShapeDtypeStruct((NP, D), jnp.float32),
)

_tc3 = pl.pallas_call(
    _tc3_body, grid=(G,),
    in_specs=[_ROWS, _ROWS, _DEGS, _BIAS],
    out_specs=_ROWS,
    out_shape=jax.ShapeDtypeStruct((NP, D), jnp.float32),
)


def kernel(x, edge_index, W1, b1, W2, b2):
    pad = EPAD - E
    src = jnp.concatenate(
        [edge_index[0], jnp.zeros((pad,), jnp.int32)]).reshape(NBLK, 1, BE)
    dst = jnp.concatenate(
        [edge_index[1], jnp.full((pad,), N, jnp.int32)]).reshape(NBLK, 1, BE)
    x_pad = jnp.pad(x, ((0, NP - N), (0, 0)))

    deg = _k_deg(dst)
    hs1 = _tc1(x_pad, W1, deg)
    agg1 = _k_agg(hs1, src, dst)
    hs2 = _tc2(agg1, hs1, deg, b1.reshape(1, D), W2)
    agg2 = _k_agg(hs2, src, dst)
    out = _tc3(agg2, hs2, deg, b2.reshape(1, D))
    return out[:N]


# speedup vs baseline: 2.0362x; 1.3482x over previous
import functools

import jax
import jax.numpy as jnp
from jax import lax
from jax.experimental import pallas as pl
from jax.experimental.pallas import tpu as pltpu

N = 10000
D = 128
E = 320000

NP = 10240
BE = 1024
NBLK = 320
EPAD = NBLK * BE

BR = 1024
G = NP // BR


def _deg_body(dst_ref, deg_ref):
    @pl.when(pl.program_id(0) == 0)
    def _():
        deg_ref[...] = jnp.zeros((NP, 1), jnp.float32)

    def body(e, _):
        d = dst_ref[0, 0, e]
        deg_ref[pl.ds(d, 1), :] = deg_ref[pl.ds(d, 1), :] + 1.0
        return 0
    lax.fori_loop(0, BE, body, 0)


_k_deg = pl.pallas_call(
    _deg_body,
    grid=(NBLK,),
    in_specs=[pl.BlockSpec((1, 1, BE), lambda i: (i, 0, 0),
                           memory_space=pltpu.SMEM)],
    out_specs=pl.BlockSpec((NP, 1), lambda i: (0, 0)),
    out_shape=jax.ShapeDtypeStruct((NP, 1), jnp.float32),
)


def _agg_body(hs_ref, src_ref, dst_ref, o0_ref, o1_ref):
    @pl.when(pl.program_id(0) == 0)
    def _():
        o0_ref[...] = jnp.zeros((NP, D), jnp.float32)
        o1_ref[...] = jnp.zeros((NP, D), jnp.float32)

    def body(t, _):
        e0 = 2 * t
        s0 = src_ref[0, 0, e0]
        d0 = dst_ref[0, 0, e0]
        s1 = src_ref[0, 0, e0 + 1]
        d1 = dst_ref[0, 0, e0 + 1]
        o0_ref[pl.ds(d0, 1), :] = (
            o0_ref[pl.ds(d0, 1), :] + hs_ref[pl.ds(s0, 1), :])
        o1_ref[pl.ds(d1, 1), :] = (
            o1_ref[pl.ds(d1, 1), :] + hs_ref[pl.ds(s1, 1), :])
        return 0
    lax.fori_loop(0, BE // 2, body, 0)


_k_agg = pl.pallas_call(
    _agg_body,
    grid=(NBLK,),
    in_specs=[pl.BlockSpec((NP, D), lambda i: (0, 0)),
              pl.BlockSpec((1, 1, BE), lambda i: (i, 0, 0),
                           memory_space=pltpu.SMEM),
              pl.BlockSpec((1, 1, BE), lambda i: (i, 0, 0),
                           memory_space=pltpu.SMEM)],
    out_specs=[pl.BlockSpec((NP, D), lambda i: (0, 0)),
               pl.BlockSpec((NP, D), lambda i: (0, 0))],
    out_shape=[jax.ShapeDtypeStruct((NP, D), jnp.float32),
               jax.ShapeDtypeStruct((NP, D), jnp.float32)],
)


def _dinv(deg_ref):
    deg = deg_ref[...] + 1.0
    return lax.rsqrt(deg)


def _tc1_body(x_ref, w_ref, deg_ref, hs_ref):
    dinv = _dinv(deg_ref)
    hs_ref[...] = dinv * jnp.dot(x_ref[...], w_ref[...],
                                 preferred_element_type=jnp.float32)


def _tc2_body(agg0_ref, agg1_ref, hs1_ref, deg_ref, b1_ref, w2_ref, hs2_ref):
    dinv = _dinv(deg_ref)
    tot = agg0_ref[...] + agg1_ref[...] + hs1_ref[...]
    h = jnp.maximum(dinv * tot + b1_ref[...], 0.0)
    hs2_ref[...] = dinv * jnp.dot(h, w2_ref[...],
                                  preferred_element_type=jnp.float32)


def _tc3_body(agg0_ref, agg1_ref, hs2_ref, deg_ref, b2_ref, out_ref):
    dinv = _dinv(deg_ref)
    tot = agg0_ref[...] + agg1_ref[...] + hs2_ref[...]
    out_ref[...] = dinv * tot + b2_ref[...]


_ROWS = pl.BlockSpec((BR, D), lambda i: (i, 0))
_FULL = pl.BlockSpec((D, D), lambda i: (0, 0))
_DEGS = pl.BlockSpec((BR, 1), lambda i: (i, 0))
_BIAS = pl.BlockSpec((1, D), lambda i: (0, 0))

_tc1 = pl.pallas_call(
    _tc1_body, grid=(G,),
    in_specs=[_ROWS, _FULL, _DEGS],
    out_specs=_ROWS,
    out_shape=jax.ShapeDtypeStruct((NP, D), jnp.float32),
)

_tc2 = pl.pallas_call(
    _tc2_body, grid=(G,),
    in_specs=[_ROWS, _ROWS, _ROWS, _DEGS, _BIAS, _FULL],
    out_specs=_ROWS,
    out_shape=jax.ShapeDtypeStruct((NP, D), jnp.float32),
)

_tc3 = pl.pallas_call(
    _tc3_body, grid=(G,),
    in_specs=[_ROWS, _ROWS, _ROWS, _DEGS, _BIAS],
    out_specs=_ROWS,
    out_shape=jax.ShapeDtypeStruct((NP, D), jnp.float32),
)


def kernel(x, edge_index, W1, b1, W2, b2):
    pad = EPAD - E
    src = jnp.concatenate(
        [edge_index[0], jnp.zeros((pad,), jnp.int32)]).reshape(NBLK, 1, BE)
    dst = jnp.concatenate(
        [edge_index[1], jnp.full((pad,), N, jnp.int32)]).reshape(NBLK, 1, BE)
    x_pad = jnp.pad(x, ((0, NP - N), (0, 0)))

    deg = _k_deg(dst)
    hs1 = _tc1(x_pad, W1, deg)
    agg1a, agg1b = _k_agg(hs1, src, dst)
    hs2 = _tc2(agg1a, agg1b, hs1, deg, b1.reshape(1, D), W2)
    agg2a, agg2b = _k_agg(hs2, src, dst)
    out = _tc3(agg2a, agg2b, hs2, deg, b2.reshape(1, D))
    return out[:N]


# speedup vs baseline: 3.2895x; 1.6156x over previous
import functools

import jax
import jax.numpy as jnp
from jax import lax
from jax.experimental import pallas as pl
from jax.experimental.pallas import tpu as pltpu

N = 10000
D = 128
E = 320000

NP = 10240
BE = 1024
NBLK = 320
EPAD = NBLK * BE

BR = 1024
G = NP // BR


def _deg_body(dst_ref, g0_ref, g1_ref):
    @pl.when(pl.program_id(0) == 0)
    def _():
        g0_ref[...] = jnp.zeros((NP, 1), jnp.float32)
        g1_ref[...] = jnp.zeros((NP, 1), jnp.float32)

    def body(t, _):
        d0 = dst_ref[0, 0, 2 * t]
        d1 = dst_ref[0, 0, 2 * t + 1]
        g0_ref[pl.ds(d0, 1), :] = g0_ref[pl.ds(d0, 1), :] + 1.0
        g1_ref[pl.ds(d1, 1), :] = g1_ref[pl.ds(d1, 1), :] + 1.0
        return 0
    lax.fori_loop(0, BE // 2, body, 0)


_k_deg = pl.pallas_call(
    _deg_body,
    grid=(NBLK,),
    in_specs=[pl.BlockSpec((1, 1, BE), lambda i: (i, 0, 0),
                           memory_space=pltpu.SMEM)],
    out_specs=[pl.BlockSpec((NP, 1), lambda i: (0, 0)),
               pl.BlockSpec((NP, 1), lambda i: (0, 0))],
    out_shape=[jax.ShapeDtypeStruct((NP, 1), jnp.float32),
               jax.ShapeDtypeStruct((NP, 1), jnp.float32)],
)


def _agg_body(hs_ref, src_ref, dst_ref, o0_ref, o1_ref, o2_ref, o3_ref):
    @pl.when(pl.program_id(0) == 0)
    def _():
        o0_ref[...] = jnp.zeros((NP, D), jnp.float32)
        o1_ref[...] = jnp.zeros((NP, D), jnp.float32)
        o2_ref[...] = jnp.zeros((NP, D), jnp.float32)
        o3_ref[...] = jnp.zeros((NP, D), jnp.float32)

    def body(t, _):
        e0 = 4 * t
        for q, o_ref in enumerate((o0_ref, o1_ref, o2_ref, o3_ref)):
            s = src_ref[0, 0, e0 + q]
            d = dst_ref[0, 0, e0 + q]
            o_ref[pl.ds(d, 1), :] = (
                o_ref[pl.ds(d, 1), :] + hs_ref[pl.ds(s, 1), :])
        return 0
    lax.fori_loop(0, BE // 4, body, 0)


_k_agg = pl.pallas_call(
    _agg_body,
    grid=(NBLK,),
    in_specs=[pl.BlockSpec((NP, D), lambda i: (0, 0)),
              pl.BlockSpec((1, 1, BE), lambda i: (i, 0, 0),
                           memory_space=pltpu.SMEM),
              pl.BlockSpec((1, 1, BE), lambda i: (i, 0, 0),
                           memory_space=pltpu.SMEM)],
    out_specs=[pl.BlockSpec((NP, D), lambda i: (0, 0))] * 4,
    out_shape=[jax.ShapeDtypeStruct((NP, D), jnp.float32)] * 4,
)


def _dinv(d0_ref, d1_ref):
    deg = d0_ref[...] + d1_ref[...] + 1.0
    return lax.rsqrt(deg)


def _tc1_body(x_ref, w_ref, d0_ref, d1_ref, hs_ref):
    dinv = _dinv(d0_ref, d1_ref)
    hs_ref[...] = dinv * jnp.dot(x_ref[...], w_ref[...],
                                 preferred_element_type=jnp.float32)


def _tc2_body(a0, a1, a2, a3, hs1_ref, d0_ref, d1_ref, b1_ref, w2_ref,
              hs2_ref):
    dinv = _dinv(d0_ref, d1_ref)
    tot = (a0[...] + a1[...]) + (a2[...] + a3[...]) + hs1_ref[...]
    h = jnp.maximum(dinv * tot + b1_ref[...], 0.0)
    hs2_ref[...] = dinv * jnp.dot(h, w2_ref[...],
                                  preferred_element_type=jnp.float32)


def _tc3_body(a0, a1, a2, a3, hs2_ref, d0_ref, d1_ref, b2_ref, out_ref):
    dinv = _dinv(d0_ref, d1_ref)
    tot = (a0[...] + a1[...]) + (a2[...] + a3[...]) + hs2_ref[...]
    out_ref[...] = dinv * tot + b2_ref[...]


_ROWS = pl.BlockSpec((BR, D), lambda i: (i, 0))
_FULL = pl.BlockSpec((D, D), lambda i: (0, 0))
_DEGS = pl.BlockSpec((BR, 1), lambda i: (i, 0))
_BIAS = pl.BlockSpec((1, D), lambda i: (0, 0))

_tc1 = pl.pallas_call(
    _tc1_body, grid=(G,),
    in_specs=[_ROWS, _FULL, _DEGS, _DEGS],
    out_specs=_ROWS,
    out_shape=jax.ShapeDtypeStruct((NP, D), jnp.float32),
)

_tc2 = pl.pallas_call(
    _tc2_body, grid=(G,),
    in_specs=[_ROWS, _ROWS, _ROWS, _ROWS, _ROWS, _DEGS, _DEGS, _BIAS, _FULL],
    out_specs=_ROWS,
    out_shape=jax.ShapeDtypeStruct((NP, D), jnp.float32),
)

_tc3 = pl.pallas_call(
    _tc3_body, grid=(G,),
    in_specs=[_ROWS, _ROWS, _ROWS, _ROWS, _ROWS, _DEGS, _DEGS, _BIAS],
    out_specs=_ROWS,
    out_shape=jax.ShapeDtypeStruct((NP, D), jnp.float32),
)


def kernel(x, edge_index, W1, b1, W2, b2):
    pad = EPAD - E
    src = jnp.concatenate(
        [edge_index[0], jnp.zeros((pad,), jnp.int32)]).reshape(NBLK, 1, BE)
    dst = jnp.concatenate(
        [edge_index[1], jnp.full((pad,), N, jnp.int32)]).reshape(NBLK, 1, BE)
    x_pad = jnp.pad(x, ((0, NP - N), (0, 0)))

    dg0, dg1 = _k_deg(dst)
    hs1 = _tc1(x_pad, W1, dg0, dg1)
    a10, a11, a12, a13 = _k_agg(hs1, src, dst)
    hs2 = _tc2(a10, a11, a12, a13, hs1, dg0, dg1, b1.reshape(1, D), W2)
    a20, a21, a22, a23 = _k_agg(hs2, src, dst)
    out = _tc3(a20, a21, a22, a23, hs2, dg0, dg1, b2.reshape(1, D))
    return out[:N]


# speedup vs baseline: 3.7176x; 1.1301x over previous
import functools

import jax
import jax.numpy as jnp
from jax import lax
from jax.experimental import pallas as pl
from jax.experimental.pallas import tpu as pltpu

N = 10000
D = 128
E = 320000

NP = 10240
BE = 1024
NBLK = 320
EPAD = NBLK * BE

BR = 1024
G = NP // BR


def _deg_body(dst_ref, g0_ref, g1_ref):
    @pl.when(pl.program_id(0) == 0)
    def _():
        g0_ref[...] = jnp.zeros((NP, 1), jnp.float32)
        g1_ref[...] = jnp.zeros((NP, 1), jnp.float32)

    def body(t, _):
        d0 = dst_ref[0, 0, 2 * t]
        d1 = dst_ref[0, 0, 2 * t + 1]
        g0_ref[pl.ds(d0, 1), :] = g0_ref[pl.ds(d0, 1), :] + 1.0
        g1_ref[pl.ds(d1, 1), :] = g1_ref[pl.ds(d1, 1), :] + 1.0
        return 0
    lax.fori_loop(0, BE // 2, body, 0)


_k_deg = pl.pallas_call(
    _deg_body,
    grid=(NBLK,),
    in_specs=[pl.BlockSpec((1, 1, BE), lambda i: (i, 0, 0),
                           memory_space=pltpu.SMEM)],
    out_specs=[pl.BlockSpec((NP, 1), lambda i: (0, 0)),
               pl.BlockSpec((NP, 1), lambda i: (0, 0))],
    out_shape=[jax.ShapeDtypeStruct((NP, 1), jnp.float32),
               jax.ShapeDtypeStruct((NP, 1), jnp.float32)],
)


NACC = 8


def _agg_body(hs_ref, src_ref, dst_ref, *o_refs):
    @pl.when(pl.program_id(0) == 0)
    def _():
        for o_ref in o_refs:
            o_ref[...] = jnp.zeros((NP, D), jnp.float32)

    def body(t, _):
        e0 = NACC * t
        for q, o_ref in enumerate(o_refs):
            s = src_ref[0, 0, e0 + q]
            d = dst_ref[0, 0, e0 + q]
            o_ref[pl.ds(d, 1), :] = (
                o_ref[pl.ds(d, 1), :] + hs_ref[pl.ds(s, 1), :])
        return 0
    lax.fori_loop(0, BE // NACC, body, 0)


_k_agg = pl.pallas_call(
    _agg_body,
    grid=(NBLK,),
    in_specs=[pl.BlockSpec((NP, D), lambda i: (0, 0)),
              pl.BlockSpec((1, 1, BE), lambda i: (i, 0, 0),
                           memory_space=pltpu.SMEM),
              pl.BlockSpec((1, 1, BE), lambda i: (i, 0, 0),
                           memory_space=pltpu.SMEM)],
    out_specs=[pl.BlockSpec((NP, D), lambda i: (0, 0))] * NACC,
    out_shape=[jax.ShapeDtypeStruct((NP, D), jnp.float32)] * NACC,
)


def _dinv(d0_ref, d1_ref):
    deg = d0_ref[...] + d1_ref[...] + 1.0
    return lax.rsqrt(deg)


def _tc1_body(x_ref, w_ref, d0_ref, d1_ref, hs_ref):
    dinv = _dinv(d0_ref, d1_ref)
    hs_ref[...] = dinv * jnp.dot(x_ref[...], w_ref[...],
                                 preferred_element_type=jnp.float32)


def _tc2_body(a0, a1, a2, a3, a4, a5, a6, a7, hs1_ref, d0_ref, d1_ref,
              b1_ref, w2_ref, hs2_ref):
    dinv = _dinv(d0_ref, d1_ref)
    tot = (((a0[...] + a1[...]) + (a2[...] + a3[...]))
           + ((a4[...] + a5[...]) + (a6[...] + a7[...]))) + hs1_ref[...]
    h = jnp.maximum(dinv * tot + b1_ref[...], 0.0)
    hs2_ref[...] = dinv * jnp.dot(h, w2_ref[...],
                                  preferred_element_type=jnp.float32)


def _tc3_body(a0, a1, a2, a3, a4, a5, a6, a7, hs2_ref, d0_ref, d1_ref,
              b2_ref, out_ref):
    dinv = _dinv(d0_ref, d1_ref)
    tot = (((a0[...] + a1[...]) + (a2[...] + a3[...]))
           + ((a4[...] + a5[...]) + (a6[...] + a7[...]))) + hs2_ref[...]
    out_ref[...] = dinv * tot + b2_ref[...]


_ROWS = pl.BlockSpec((BR, D), lambda i: (i, 0))
_FULL = pl.BlockSpec((D, D), lambda i: (0, 0))
_DEGS = pl.BlockSpec((BR, 1), lambda i: (i, 0))
_BIAS = pl.BlockSpec((1, D), lambda i: (0, 0))

_tc1 = pl.pallas_call(
    _tc1_body, grid=(G,),
    in_specs=[_ROWS, _FULL, _DEGS, _DEGS],
    out_specs=_ROWS,
    out_shape=jax.ShapeDtypeStruct((NP, D), jnp.float32),
)

_tc2 = pl.pallas_call(
    _tc2_body, grid=(G,),
    in_specs=[_ROWS] * 9 + [_DEGS, _DEGS, _BIAS, _FULL],
    out_specs=_ROWS,
    out_shape=jax.ShapeDtypeStruct((NP, D), jnp.float32),
)

_tc3 = pl.pallas_call(
    _tc3_body, grid=(G,),
    in_specs=[_ROWS] * 9 + [_DEGS, _DEGS, _BIAS],
    out_specs=_ROWS,
    out_shape=jax.ShapeDtypeStruct((NP, D), jnp.float32),
)


def kernel(x, edge_index, W1, b1, W2, b2):
    pad = EPAD - E
    src = jnp.concatenate(
        [edge_index[0], jnp.zeros((pad,), jnp.int32)]).reshape(NBLK, 1, BE)
    dst = jnp.concatenate(
        [edge_index[1], jnp.full((pad,), N, jnp.int32)]).reshape(NBLK, 1, BE)
    x_pad = jnp.pad(x, ((0, NP - N), (0, 0)))

    dg0, dg1 = _k_deg(dst)
    hs1 = _tc1(x_pad, W1, dg0, dg1)
    agg1 = _k_agg(hs1, src, dst)
    hs2 = _tc2(*agg1, hs1, dg0, dg1, b1.reshape(1, D), W2)
    agg2 = _k_agg(hs2, src, dst)
    out = _tc3(*agg2, hs2, dg0, dg1, b2.reshape(1, D))
    return out[:N]
